# baseline (speedup 1.0000x reference)
import functools

import jax
import jax.numpy as jnp
from jax import lax
from jax.experimental import pallas as pl
from jax.experimental.pallas import tpu as pltpu
from jax.experimental.pallas import tpu_sc as plsc

_N = 524288
_NC = 2
_NS = 16
_L = 16
_UNROLL = 8

_A_PER_TILE = 9728
_A_CHUNK = _A_PER_TILE // 2
_A_TOTAL = _A_PER_TILE * _NC * _NS
_B_PER_TILE = (_N - _A_TOTAL) // _NS
_B_CHUNK = _B_PER_TILE // 2


def _clamp_loop(buf, off, count, lo, hi):
    def body(i, carry):
        o = off + i * (_L * _UNROLL)
        for j in range(_UNROLL):
            s = pl.ds(o + j * _L, _L)
            buf[s] = jnp.maximum(jnp.minimum(buf[s], hi), lo)
        return carry

    lax.fori_loop(0, count // (_L * _UNROLL), body, 0)


def _build_sc_clamp():
    mesh = plsc.VectorSubcoreMesh(core_axis_name="c", subcore_axis_name="s")

    @functools.partial(
        pl.kernel,
        mesh=mesh,
        out_type=jax.ShapeDtypeStruct((_N,), jnp.float32),
        scratch_types=[
            pltpu.VMEM((_A_PER_TILE + _B_PER_TILE,), jnp.float32),
            pltpu.VMEM((64,), jnp.float32),
            pltpu.SemaphoreType.DMA,
            pltpu.SemaphoreType.DMA,
            pltpu.SemaphoreType.DMA,
            pltpu.SemaphoreType.DMA,
            pltpu.SemaphoreType.DMA,
            pltpu.SemaphoreType.DMA,
        ],
    )
    def sc_clamp(x_hbm, vals_hbm, out_hbm, buf, vals_v,
                 sa0, sa1, sb0, sb1, vsem, osem):
        cid = lax.axis_index("c")
        wid = lax.axis_index("s") * _NC + cid
        a_base = wid * _A_PER_TILE
        b_base = _A_TOTAL + lax.axis_index("s") * _B_PER_TILE
        b_off = _A_PER_TILE

        vcopy = pltpu.async_copy(vals_hbm, vals_v, vsem)
        in_a = []
        for k in range(2):
            in_a.append(pltpu.async_copy(
                x_hbm.at[pl.ds(a_base + k * _A_CHUNK, _A_CHUNK)],
                buf.at[pl.ds(k * _A_CHUNK, _A_CHUNK)], (sa0, sa1)[k]))

        @pl.when(cid == 1)
        def _():
            for k in range(2):
                pltpu.async_copy(
                    x_hbm.at[pl.ds(b_base + k * _B_CHUNK, _B_CHUNK)],
                    buf.at[pl.ds(b_off + k * _B_CHUNK, _B_CHUNK)],
                    (sb0, sb1)[k])

        vcopy.wait()
        lo = jnp.full((_L,), vals_v[pl.ds(0, _L)][0], jnp.float32)
        hi = jnp.full((_L,), vals_v[pl.ds(48, _L)][_L - 1], jnp.float32)

        out_a = []
        for k in range(2):
            in_a[k].wait()
            _clamp_loop(buf, k * _A_CHUNK, _A_CHUNK, lo, hi)
            out_a.append(pltpu.async_copy(
                buf.at[pl.ds(k * _A_CHUNK, _A_CHUNK)],
                out_hbm.at[pl.ds(a_base + k * _A_CHUNK, _A_CHUNK)], osem))

        @pl.when(cid == 1)
        def _():
            out_b = []
            for k in range(2):
                pltpu.make_async_copy(
                    x_hbm.at[pl.ds(b_base + k * _B_CHUNK, _B_CHUNK)],
                    buf.at[pl.ds(b_off + k * _B_CHUNK, _B_CHUNK)],
                    (sb0, sb1)[k]).wait()
                _clamp_loop(buf, b_off + k * _B_CHUNK, _B_CHUNK, lo, hi)
                out_b.append(pltpu.async_copy(
                    buf.at[pl.ds(b_off + k * _B_CHUNK, _B_CHUNK)],
                    out_hbm.at[pl.ds(b_base + k * _B_CHUNK, _B_CHUNK)], osem))
            for cp in out_b:
                cp.wait()

        for cp in out_a:
            cp.wait()

    return sc_clamp


_sc_clamp = _build_sc_clamp()


def kernel(tensor_input, values):
    out = _sc_clamp(tensor_input.reshape(_N), values)
    return out.reshape(tensor_input.shape)
